# baseline (speedup 1.0000x reference)
import functools

import jax
import jax.numpy as jnp
from jax.experimental import pallas as pl

N = 2048
BLK = 256


def _propagate_body(t_ref, ab_ref, out_ref):
    i = pl.program_id(0)
    t = t_ref[...]
    c = jnp.dot(t, ab_ref[...], preferred_element_type=jnp.int32)
    rows = i * BLK + jax.lax.broadcasted_iota(jnp.int32, (BLK, N), 0)
    cols = jax.lax.broadcasted_iota(jnp.int32, (BLK, N), 1)
    keep = ((c > 0) & (rows != cols)) | (t != 0)
    out_ref[...] = jnp.where(keep, 1, 0).astype(jnp.int8)


def _propagate(t, ab):
    return pl.pallas_call(
        _propagate_body,
        grid=(N // BLK,),
        in_specs=[
            pl.BlockSpec((BLK, N), lambda i: (i, 0)),
            pl.BlockSpec((N, N), lambda i: (0, 0)),
        ],
        out_specs=pl.BlockSpec((BLK, N), lambda i: (i, 0)),
        out_shape=jax.ShapeDtypeStruct((N, N), jnp.int8),
    )(t, ab)


def _layer_body(has_linear, relu, weight,
                xi_ref, t_ref, wt_ref, b_ref, wgt_ref, bg_ref, out_ref):
    xi = xi_ref[...]
    if has_linear:
        xlin = jnp.dot(xi, wt_ref[...], preferred_element_type=jnp.float32)
        xlin = xlin + b_ref[...]
    else:
        xlin = xi
    t = t_ref[...].astype(jnp.float32)
    deg = jnp.sum(t, axis=1, keepdims=True) + 1.0
    dinv = jax.lax.rsqrt(deg)
    y = jnp.dot(xlin, wgt_ref[...], preferred_element_type=jnp.float32) * dinv
    agg = jnp.dot(t, y, preferred_element_type=jnp.float32) + y
    g = agg * dinv + bg_ref[...]
    if relu:
        g = jnp.maximum(g, 0.0)
    out_ref[...] = xlin + weight * g


def _layer(xi, t, wt, b, wgt, bg, has_linear, relu, weight):
    body = functools.partial(_layer_body, has_linear, relu, weight)
    return pl.pallas_call(
        body,
        out_shape=jax.ShapeDtypeStruct((N, wgt.shape[1]), jnp.float32),
    )(xi, t, wt, b, wgt, bg)


def kernel(x, A_neg, A_pos, W1, b1, W2, b2, W3, b3,
           Wg1, bg1, Wg2, bg2, Wg3, bg3, Wg4, bg4, Wg5, bg5, Wg6, bg6):
    t = A_neg.T.astype(jnp.int8)
    ab = A_pos.T.astype(jnp.int8)

    w1t = W1.T
    b1r = b1[None, :]
    w2t = jnp.pad(W2.T, ((0, 0), (0, 2)))
    b2r = jnp.pad(b2, (0, 2))[None, :]
    w3t = jnp.pad(W3.T, ((0, 2), (0, 0)))
    b3r = b3[None, :]
    wg1t = Wg1.T
    bg1r = bg1[None, :]
    wg2t = jnp.pad(Wg2.T, ((0, 2), (0, 2)))
    bg2r = jnp.pad(bg2, (0, 2))[None, :]
    wg3t, bg3r = Wg3.T, bg3[None, :]
    wg4t, bg4r = Wg4.T, bg4[None, :]
    wg5t, bg5r = Wg5.T, bg5[None, :]
    wg6t, bg6r = Wg6.T, bg6[None, :]

    x1 = _layer(x, t, w1t, b1r, wg1t, bg1r, True, True, 1.0)
    t = _propagate(t, ab)
    x2 = _layer(x1, t, w2t, b2r, wg2t, bg2r, True, True, 1.0)
    t = _propagate(t, ab)
    x3 = _layer(x2, t, w3t, b3r, wg3t, bg3r, True, True, 0.5)
    t = _propagate(t, ab)
    x4 = _layer(x3, t, x3, b3r, wg4t, bg4r, False, True, 0.5)
    t = _propagate(t, ab)
    x5 = _layer(x4, t, x4, b3r, wg5t, bg5r, False, True, 0.25)
    t = _propagate(t, ab)
    x6 = _layer(x5, t, x5, b3r, wg6t, bg6r, False, False, 0.25)
    return x6


# speedup vs baseline: 1.1041x; 1.1041x over previous
import functools

import jax
import jax.numpy as jnp
from jax.experimental import pallas as pl

N = 2048
BLK = 256


def _propagate_body(t_ref, ab_ref, out_ref):
    i = pl.program_id(0)
    t = t_ref[...]
    c = jnp.dot(t, ab_ref[...], preferred_element_type=jnp.float32)
    rows = i * BLK + jax.lax.broadcasted_iota(jnp.int32, (BLK, N), 0)
    cols = jax.lax.broadcasted_iota(jnp.int32, (BLK, N), 1)
    keep = ((c > 0) & (rows != cols)) | (t > 0)
    out_ref[...] = jnp.where(keep, 1.0, 0.0).astype(jnp.bfloat16)


def _propagate(t, ab):
    return pl.pallas_call(
        _propagate_body,
        grid=(N // BLK,),
        in_specs=[
            pl.BlockSpec((BLK, N), lambda i: (i, 0)),
            pl.BlockSpec((N, N), lambda i: (0, 0)),
        ],
        out_specs=pl.BlockSpec((BLK, N), lambda i: (i, 0)),
        out_shape=jax.ShapeDtypeStruct((N, N), jnp.bfloat16),
    )(t, ab)


def _layer_body(has_linear, relu, weight,
                xi_ref, t_ref, wt_ref, b_ref, wgt_ref, bg_ref, out_ref):
    xi = xi_ref[...]
    if has_linear:
        xlin = jnp.dot(xi, wt_ref[...], preferred_element_type=jnp.float32)
        xlin = xlin + b_ref[...]
    else:
        xlin = xi
    t = t_ref[...]
    deg = jnp.sum(t.astype(jnp.float32), axis=1, keepdims=True) + 1.0
    dinv = jax.lax.rsqrt(deg)
    y = jnp.dot(xlin, wgt_ref[...], preferred_element_type=jnp.float32) * dinv
    y_hi = y.astype(jnp.bfloat16)
    y_lo = (y - y_hi.astype(jnp.float32)).astype(jnp.bfloat16)
    agg = (jnp.dot(t, y_hi, preferred_element_type=jnp.float32)
           + jnp.dot(t, y_lo, preferred_element_type=jnp.float32) + y)
    g = agg * dinv + bg_ref[...]
    if relu:
        g = jnp.maximum(g, 0.0)
    out_ref[...] = xlin + weight * g


def _layer(xi, t, wt, b, wgt, bg, has_linear, relu, weight):
    body = functools.partial(_layer_body, has_linear, relu, weight)
    return pl.pallas_call(
        body,
        out_shape=jax.ShapeDtypeStruct((N, wgt.shape[1]), jnp.float32),
    )(xi, t, wt, b, wgt, bg)


def kernel(x, A_neg, A_pos, W1, b1, W2, b2, W3, b3,
           Wg1, bg1, Wg2, bg2, Wg3, bg3, Wg4, bg4, Wg5, bg5, Wg6, bg6):
    t = A_neg.T.astype(jnp.bfloat16)
    ab = A_pos.T.astype(jnp.bfloat16)

    w1t = W1.T
    b1r = b1[None, :]
    w2t = jnp.pad(W2.T, ((0, 0), (0, 2)))
    b2r = jnp.pad(b2, (0, 2))[None, :]
    w3t = jnp.pad(W3.T, ((0, 2), (0, 0)))
    b3r = b3[None, :]
    wg1t = Wg1.T
    bg1r = bg1[None, :]
    wg2t = jnp.pad(Wg2.T, ((0, 2), (0, 2)))
    bg2r = jnp.pad(bg2, (0, 2))[None, :]
    wg3t, bg3r = Wg3.T, bg3[None, :]
    wg4t, bg4r = Wg4.T, bg4[None, :]
    wg5t, bg5r = Wg5.T, bg5[None, :]
    wg6t, bg6r = Wg6.T, bg6[None, :]

    x1 = _layer(x, t, w1t, b1r, wg1t, bg1r, True, True, 1.0)
    t = _propagate(t, ab)
    x2 = _layer(x1, t, w2t, b2r, wg2t, bg2r, True, True, 1.0)
    t = _propagate(t, ab)
    x3 = _layer(x2, t, w3t, b3r, wg3t, bg3r, True, True, 0.5)
    t = _propagate(t, ab)
    x4 = _layer(x3, t, x3, b3r, wg4t, bg4r, False, True, 0.5)
    t = _propagate(t, ab)
    x5 = _layer(x4, t, x4, b3r, wg5t, bg5r, False, True, 0.25)
    t = _propagate(t, ab)
    x6 = _layer(x5, t, x5, b3r, wg6t, bg6r, False, False, 0.25)
    return x6


# speedup vs baseline: 1.3468x; 1.2198x over previous
import jax
import jax.numpy as jnp
from jax.experimental import pallas as pl
from jax.experimental.pallas import tpu as pltpu

N = 2048
BLK = 256


def _mega_body(x_ref, t0_ref, ab_ref,
               w1t_ref, b1_ref, w2t_ref, b2_ref, w3t_ref, b3_ref,
               wg1t_ref, bg1_ref, wg2t_ref, bg2_ref, wg3t_ref, bg3_ref,
               wg4t_ref, bg4_ref, wg5t_ref, bg5_ref, wg6t_ref, bg6_ref,
               out_ref, tscr_ref, deg_ref):
    f32 = jnp.float32
    bf16 = jnp.bfloat16

    def gcn(xlin, t_ref_cur, deg, wgt_ref, bg_ref, use_relu, w):
        t = t_ref_cur[...]
        dinv = jax.lax.rsqrt(deg)
        y = jnp.dot(xlin, wgt_ref[...], preferred_element_type=f32) * dinv
        y_hi = y.astype(bf16)
        y_lo = (y - y_hi.astype(f32)).astype(bf16)
        agg = (jnp.dot(t, y_hi, preferred_element_type=f32)
               + jnp.dot(t, y_lo, preferred_element_type=f32) + y)
        g = agg * dinv + bg_ref[...]
        if use_relu:
            g = jnp.maximum(g, 0.0)
        return xlin + w * g

    def prop(src_ref, dst_ref):
        cols = jax.lax.broadcasted_iota(jnp.int32, (BLK, N), 1)

        def body(blk, carry):
            sl = pl.ds(blk * BLK, BLK)
            t = src_ref[sl, :]
            c = jnp.dot(t, ab_ref[...], preferred_element_type=f32)
            rows = blk * BLK + jax.lax.broadcasted_iota(
                jnp.int32, (BLK, N), 0)
            keep = ((c > 0.0) & (rows != cols)) | (t > 0)
            tn = jnp.where(keep, 1.0, 0.0)
            dst_ref[sl, :] = tn.astype(bf16)
            deg_ref[sl, :] = jnp.sum(tn, axis=1, keepdims=True) + 1.0
            return carry

        jax.lax.fori_loop(0, N // BLK, body, 0)

    t0 = t0_ref[...]
    deg0 = jnp.sum(t0.astype(f32), axis=1, keepdims=True) + 1.0
    x1lin = (jnp.dot(x_ref[...], w1t_ref[...], preferred_element_type=f32)
             + b1_ref[...])
    x1 = gcn(x1lin, t0_ref, deg0, wg1t_ref, bg1_ref, True, 1.0)
    prop(t0_ref, tscr_ref)
    x2lin = (jnp.dot(x1, w2t_ref[...], preferred_element_type=f32)
             + b2_ref[...])
    x2 = gcn(x2lin, tscr_ref, deg_ref[...], wg2t_ref, bg2_ref, True, 1.0)
    prop(tscr_ref, tscr_ref)
    x3lin = (jnp.dot(x2, w3t_ref[...], preferred_element_type=f32)
             + b3_ref[...])
    x3 = gcn(x3lin, tscr_ref, deg_ref[...], wg3t_ref, bg3_ref, True, 0.5)
    prop(tscr_ref, tscr_ref)
    x4 = gcn(x3, tscr_ref, deg_ref[...], wg4t_ref, bg4_ref, True, 0.5)
    prop(tscr_ref, tscr_ref)
    x5 = gcn(x4, tscr_ref, deg_ref[...], wg5t_ref, bg5_ref, True, 0.25)
    prop(tscr_ref, tscr_ref)
    x6 = gcn(x5, tscr_ref, deg_ref[...], wg6t_ref, bg6_ref, False, 0.25)
    out_ref[...] = x6


def kernel(x, A_neg, A_pos, W1, b1, W2, b2, W3, b3,
           Wg1, bg1, Wg2, bg2, Wg3, bg3, Wg4, bg4, Wg5, bg5, Wg6, bg6):
    t0 = A_neg.T.astype(jnp.bfloat16)
    ab = A_pos.T.astype(jnp.bfloat16)

    w1t = W1.T
    b1r = b1[None, :]
    w2t = jnp.pad(W2.T, ((0, 0), (0, 2)))
    b2r = jnp.pad(b2, (0, 2))[None, :]
    w3t = jnp.pad(W3.T, ((0, 2), (0, 0)))
    b3r = b3[None, :]
    wg1t = Wg1.T
    bg1r = bg1[None, :]
    wg2t = jnp.pad(Wg2.T, ((0, 2), (0, 2)))
    bg2r = jnp.pad(bg2, (0, 2))[None, :]
    wg3t, bg3r = Wg3.T, bg3[None, :]
    wg4t, bg4r = Wg4.T, bg4[None, :]
    wg5t, bg5r = Wg5.T, bg5[None, :]
    wg6t, bg6r = Wg6.T, bg6[None, :]

    return pl.pallas_call(
        _mega_body,
        out_shape=jax.ShapeDtypeStruct((N, 64), jnp.float32),
        scratch_shapes=[pltpu.VMEM((N, N), jnp.bfloat16),
                        pltpu.VMEM((N, 1), jnp.float32)],
    )(x, t0, ab, w1t, b1r, w2t, b2r, w3t, b3r,
      wg1t, bg1r, wg2t, bg2r, wg3t, bg3r,
      wg4t, bg4r, wg5t, bg5r, wg6t, bg6r)
